# Initial kernel scaffold; baseline (speedup 1.0000x reference)
import jax
import jax.numpy as jnp
from jax.experimental import pallas as pl


def kernel(x0, dw, u):
    raise NotImplementedError("write your pallas kernel here")


# speedup vs baseline: 17.1200x; 17.1200x over previous
import functools

import jax
import jax.numpy as jnp
import numpy as np
from jax.experimental import pallas as pl
from jax.experimental.pallas import tpu as pltpu

Dim = 2
R = 1.0
sigma = float(np.sqrt(2.0))
gamma = 1.0
NSTEP = 50
total_time = 0.2
dt = total_time / NSTEP
E1 = float(np.exp(-gamma * dt).astype(np.float32))

LANES = 128
RB = 8
BLK = RB * LANES


def _rho_tile(xe0, xe1, dr0, dr1, df0, df1):
    a = dr0 * dr0 + dr1 * dr1
    b = 2.0 * (dr0 * df0 + dr1 * df1)
    c = (2.0 * dr0 * xe0 + df0 * df0) + (2.0 * dr1 * xe1 + df1 * df1)
    d = 2.0 * (df0 * xe0 + df1 * xe1)
    e = (xe0 * xe0 + xe1 * xe1) - R ** 2
    p = (8.0 * a * c - 3.0 * (b * b)) / (8.0 * (a * a))
    q = (b * b * b - 4.0 * a * b * c + 8.0 * (a * a) * d) / (8.0 * (a * a * a))
    sign_q = jnp.sign(q)
    D0 = c * c - 3.0 * b * d + 12.0 * a * e
    D1 = (2.0 * (c * c * c) - 9.0 * b * c * d + 27.0 * (b * b) * e
          + 27.0 * a * (d * d) - 72.0 * a * c * e)
    D2 = D1 * D1 - 4.0 * (D0 * D0 * D0)
    signal_D2 = jnp.ceil((jnp.sign(D2) + 1.0) / 2.0)
    QQ = (D1 + jnp.sqrt(jnp.abs(D2))) / 2.0
    Q = jnp.sign(QQ) * jnp.abs(QQ) ** (1.0 / 3.0)
    S_plus = 0.5 * jnp.sqrt(jnp.abs((Q + D0 / Q) / (3.0 * a) - 2.0 * p / 3.0))
    m = jnp.minimum(jnp.sqrt(jnp.abs(D1 * D1 / 4.0 / (D0 * D0 * D0))), 1.0)
    phi = jax.lax.atan2(jnp.sqrt((1.0 - m) * (m + 1.0)), m)
    S_minus = 0.5 * jnp.sqrt(jnp.abs(2.0 * jnp.sqrt(jnp.abs(D0)) * jnp.cos(phi / 3.0) / (3.0 * a)
                                     - 2.0 * p / 3.0))
    S = signal_D2 * S_plus + (1.0 - signal_D2) * S_minus
    temp = -4.0 * (S * S) - 2.0 * p + jnp.abs(q / S)
    sqrt_rho = 0.5 * jnp.sqrt(jnp.abs(temp)) - b / (4.0 * a) - sign_q * S
    bad = (1.0 - sqrt_rho) * sqrt_rho < 0.0
    new_temp = -4.0 * (S * S) - 2.0 * p - jnp.abs(q / S)
    new_sqrt_rho = 0.5 * jnp.sqrt(jnp.abs(new_temp)) - b / (4.0 * a) + sign_q * S
    sqrt_rho_final = jnp.where(bad, new_sqrt_rho, sqrt_rho)
    return sqrt_rho_final * sqrt_rho_final, jnp.abs(sqrt_rho_final)


def _body(x0_ref, dwt_ref, u_ref, y_ref):
    uf = u_ref[0, 0]
    x00 = x0_ref[0]
    x01 = x0_ref[1]

    d00 = sigma * dwt_ref[0, 0]
    d01 = sigma * dwt_ref[1, 0]
    g0 = uf * x00 * dt
    g1 = uf * x01 * dt
    t0 = x00 + (g0 + d00)
    t1 = x01 + (g1 + d01)
    exit0 = t0 * t0 + t1 * t1 >= R * R
    w0 = (uf * uf + 2.0) * (x00 * x00 + x01 * x01) - 2.0 * Dim

    zeros = jnp.zeros_like(x00)

    def step(t, st):
        x0_, x1_, alivef, exitedf, A, ep, epk, xs0, xs1, sdf0, sdf1 = st
        alive = alivef > 0.0
        d0 = sigma * dwt_ref[0, t]
        d1 = sigma * dwt_ref[1, t]
        g0 = uf * x0_ * dt
        g1 = uf * x1_ * dt
        t0 = x0_ + (g0 + d0)
        t1 = x1_ + (g1 + d1)
        ex = t0 * t0 + t1 * t1 >= R * R
        nx0 = (x0_ + g0) + d0
        nx1 = (x1_ + g1) + d1
        newexit = alive & ex
        xs0 = jnp.where(newexit, x0_, xs0)
        xs1 = jnp.where(newexit, x1_, xs1)
        sdf0 = jnp.where(newexit, d0, sdf0)
        sdf1 = jnp.where(newexit, d1, sdf1)
        epk = jnp.where(newexit, ep, epk)
        exitedf = jnp.where(newexit, 1.0, exitedf)
        aliveN = alive & jnp.logical_not(ex)
        x0_ = jnp.where(aliveN, nx0, x0_)
        x1_ = jnp.where(aliveN, nx1, x1_)
        w = (uf * uf + 2.0) * (x0_ * x0_ + x1_ * x1_) - 2.0 * Dim
        A = A + jnp.where(aliveN, w * ep, 0.0)
        ep = ep * E1
        alivef = jnp.where(aliveN, 1.0, 0.0)
        return (x0_, x1_, alivef, exitedf, A, ep, epk, xs0, xs1, sdf0, sdf1)

    st = (x00, x01, jnp.ones_like(x00), zeros,
          zeros, jnp.ones_like(x00), zeros, zeros, zeros, zeros, zeros)
    st = jax.lax.fori_loop(1, NSTEP, step, st, unroll=2)
    x0_, x1_, alivef, exitedf, A, ep, epk, xs0, xs1, sdf0, sdf1 = st
    exited = exitedf > 0.0

    xs0 = jnp.where(exited, xs0, x0_)
    xs1 = jnp.where(exited, xs1, x1_)
    epk = jnp.where(exited, epk, ep)
    xe0 = jnp.where(exited, xs0, 0.1)
    xe1 = jnp.where(exited, xs1, 0.1)
    dr0 = jnp.where(exited, uf * xs0 * dt, 0.01)
    dr1 = jnp.where(exited, uf * xs1 * dt, 0.01)
    df0 = jnp.where(exited, sdf0, 0.01)
    df1 = jnp.where(exited, sdf1, 0.01)
    rho_q, srho_q = _rho_tile(xe0, xe1, dr0, dr1, df0, df1)
    rho = jnp.where(exited, rho_q, 0.0)
    srho = jnp.where(exited, srho_q, 0.0)
    xk0 = xs0 + uf * xs0 * dt * rho + sdf0 * srho
    xk1 = xs1 + uf * xs1 * dt * rho + sdf1 * srho
    nrm = xk0 * xk0 + xk1 * xk1
    term = epk * (rho * ((uf * uf + 2.0) * nrm - 2.0 * Dim) * dt
                  + jnp.exp(-gamma * dt * rho) * nrm)

    xe0b = jnp.where(exit0, x00, 0.1)
    xe1b = jnp.where(exit0, x01, 0.1)
    dr0b = jnp.where(exit0, uf * x00 * dt, 0.01)
    dr1b = jnp.where(exit0, uf * x01 * dt, 0.01)
    df0b = jnp.where(exit0, d00, 0.01)
    df1b = jnp.where(exit0, d01, 0.01)
    rho0, _ = _rho_tile(xe0b, xe1b, dr0b, dr1b, df0b, df1b)
    coef0 = jnp.where(exit0, rho0, 1.0)
    D0x = jnp.exp(-gamma * dt * coef0)
    y_ref[...] = coef0 * w0 * dt + D0x * (dt * A + term)


@jax.jit
def kernel(x0, dw, u):
    nsamp = x0.shape[0]
    nblk = nsamp // BLK
    rows = nsamp // LANES
    dwt = dw.transpose(1, 2, 0).reshape(2, NSTEP, rows, LANES)
    x0r = x0.transpose(1, 0).reshape(2, rows, LANES)
    u2d = jnp.reshape(u.astype(jnp.float32), (1, 1))

    y = pl.pallas_call(
        _body,
        grid=(nblk,),
        in_specs=[
            pl.BlockSpec((2, RB, LANES), lambda i: (0, i, 0)),
            pl.BlockSpec((2, NSTEP, RB, LANES), lambda i: (0, 0, i, 0)),
            pl.BlockSpec(memory_space=pltpu.SMEM),
        ],
        out_specs=pl.BlockSpec((RB, LANES), lambda i: (i, 0)),
        out_shape=jax.ShapeDtypeStruct((rows, LANES), jnp.float32),
    )(x0r, dwt, u2d)
    return y.reshape(nsamp, 1)
